# baseline (speedup 1.0000x reference)
import jax
import jax.numpy as jnp
from jax import lax
from jax.experimental import pallas as pl
from jax.experimental.pallas import tpu as pltpu

N_HOT = 32
RANK = 32
TOPK = 8
CAT = TOPK * RANK


def _route_body(topk_ref, w_ref, ls_ref, lb_ref, idx_ref, sv_ref, bv_ref):
    wv = w_ref[0:1, 0:N_HOT]
    hs = jnp.sum(wv)
    wn = jnp.where(hs > 1e-8, wv / hs, wv)
    cols = lax.broadcasted_iota(jnp.int32, (1, N_HOT), 1)
    cols8 = lax.broadcasted_iota(jnp.int32, (1, TOPK), 1)
    eff_k = jnp.minimum(topk_ref[0], N_HOT)
    cur = wn
    tw = []
    idxrow = jnp.zeros((1, TOPK), jnp.int32)
    for j in range(TOPK):
        m = jnp.max(cur)
        am = jnp.min(jnp.where(cur == m, cols, N_HOT))
        tw.append(jnp.where(j < eff_k, m, 0.0))
        cur = jnp.where(cols == am, -1.0, cur)
        idxrow = jnp.where(cols8 == j, am, idxrow)
    idx_ref[...] = idxrow
    s = sum(tw) + 1e-8
    for j in range(TOPK):
        wjn = tw[j] / s
        sv_ref[0:1, j * RANK:(j + 1) * RANK] = wjn * ls_ref[0:1, :]
        bv_ref[0:1, j * RANK:(j + 1) * RANK] = wjn * lb_ref[0:1, :]


def _main_body(idx_ref, *refs):
    a = refs[0:TOPK]
    b = refs[TOPK:2 * TOPK]
    sv_ref = refs[2 * TOPK]
    bv_ref = refs[2 * TOPK + 1]
    x_ref = refs[2 * TOPK + 2]
    o_ref = refs[2 * TOPK + 3]
    acatt = refs[2 * TOPK + 4]
    bcat = refs[2 * TOPK + 5]
    t = pl.program_id(0)

    @pl.when(t == 0)
    def _concat():
        for j in range(TOPK):
            acatt[j * RANK:(j + 1) * RANK, :] = a[j][0].astype(jnp.bfloat16)
            bcat[j * RANK:(j + 1) * RANK, :] = b[j][0].astype(jnp.bfloat16)

    xb = x_ref[...].astype(jnp.bfloat16)
    u = lax.dot_general(xb, acatt[...], (((1,), (1,)), ((), ())),
                        preferred_element_type=jnp.float32)
    u = u * sv_ref[0:1, :] + bv_ref[0:1, :]
    o_ref[...] = jnp.dot(u.astype(jnp.bfloat16), bcat[...],
                         preferred_element_type=jnp.float32)


def kernel(x, weights, A_hot, B_hot, latent_scale, latent_bias, top_k):
    batch, seq, d_in = x.shape
    d_out = B_hot.shape[-1]
    n_tok = batch * seq
    x_flat = x.reshape(n_tok, d_in)
    a_t = jnp.transpose(A_hot, (0, 2, 1))

    idx8, svec, bvec = pl.pallas_call(
        _route_body,
        in_specs=[
            pl.BlockSpec(memory_space=pltpu.SMEM),
            pl.BlockSpec((1, weights.shape[0]), lambda: (0, 0)),
            pl.BlockSpec((1, RANK), lambda: (0, 0)),
            pl.BlockSpec((1, RANK), lambda: (0, 0)),
        ],
        out_specs=[
            pl.BlockSpec((1, TOPK), lambda: (0, 0)),
            pl.BlockSpec((1, CAT), lambda: (0, 0)),
            pl.BlockSpec((1, CAT), lambda: (0, 0)),
        ],
        out_shape=[
            jax.ShapeDtypeStruct((1, TOPK), jnp.int32),
            jax.ShapeDtypeStruct((1, CAT), jnp.float32),
            jax.ShapeDtypeStruct((1, CAT), jnp.float32),
        ],
    )(
        jnp.asarray(top_k, jnp.int32).reshape(1),
        weights.reshape(1, -1),
        latent_scale.reshape(1, -1),
        latent_bias.reshape(1, -1),
    )

    bt = 512
    grid = (n_tok // bt,)
    a_specs = [
        pl.BlockSpec((1, RANK, d_in),
                     (lambda j: (lambda t, idx: (idx[0, j], 0, 0)))(j))
        for j in range(TOPK)
    ]
    b_specs = [
        pl.BlockSpec((1, RANK, d_out),
                     (lambda j: (lambda t, idx: (idx[0, j], 0, 0)))(j))
        for j in range(TOPK)
    ]
    grid_spec = pltpu.PrefetchScalarGridSpec(
        num_scalar_prefetch=1,
        grid=grid,
        in_specs=a_specs + b_specs + [
            pl.BlockSpec((1, CAT), lambda t, idx: (0, 0)),
            pl.BlockSpec((1, CAT), lambda t, idx: (0, 0)),
            pl.BlockSpec((bt, d_in), lambda t, idx: (t, 0)),
        ],
        out_specs=pl.BlockSpec((bt, d_out), lambda t, idx: (t, 0)),
        scratch_shapes=[
            pltpu.VMEM((CAT, d_in), jnp.bfloat16),
            pltpu.VMEM((CAT, d_out), jnp.bfloat16),
        ],
    )
    out = pl.pallas_call(
        _main_body,
        grid_spec=grid_spec,
        out_shape=jax.ShapeDtypeStruct((n_tok, d_out), jnp.float32),
    )(
        idx8,
        *([a_t] * TOPK),
        *([B_hot] * TOPK),
        svec,
        bvec,
        x_flat,
    )
    return out.reshape(batch, seq, d_out)


# speedup vs baseline: 1.0564x; 1.0564x over previous
import jax
import jax.numpy as jnp
from jax import lax
from jax.experimental import pallas as pl
from jax.experimental.pallas import tpu as pltpu

N_HOT = 32
RANK = 32
TOPK = 8
CAT = TOPK * RANK


def _body(topk_ref, w_ref, lst_ref, lb_ref, x_ref, a_hbm, b_hbm, o_ref,
          land_a, land_b, acatt, bcat, bv, sems):
    t = pl.program_id(0)

    @pl.when(t == 0)
    def _route_gather_concat():
        wv = w_ref[0:1, 0:N_HOT]
        hs = jnp.sum(wv)
        wn = jnp.where(hs > 1e-8, wv / hs, wv)
        cols = lax.broadcasted_iota(jnp.int32, (1, N_HOT), 1)
        eff_k = jnp.minimum(topk_ref[0], N_HOT)
        cur = wn
        tw = []
        copies = []
        for j in range(TOPK):
            m = jnp.max(cur)
            am = jnp.min(jnp.where(cur == m, cols, N_HOT))
            tw.append(jnp.where(j < eff_k, m, 0.0))
            cur = jnp.where(cols == am, -1.0, cur)
            ca = pltpu.make_async_copy(
                a_hbm.at[am], land_a.at[pl.ds(j * RANK, RANK), :],
                sems.at[2 * j])
            cb = pltpu.make_async_copy(
                b_hbm.at[am], land_b.at[pl.ds(j * RANK, RANK), :],
                sems.at[2 * j + 1])
            ca.start()
            cb.start()
            copies.append(ca)
            copies.append(cb)
        s = sum(tw) + 1e-8
        for c in copies:
            c.wait()
        for j in range(TOPK):
            wjn = tw[j] / s
            bv[0:1, j * RANK:(j + 1) * RANK] = wjn * lb_ref[0:1, :]
            acatt[j * RANK:(j + 1) * RANK, :] = (
                land_a[j * RANK:(j + 1) * RANK, :]
                * (wjn * lst_ref[...])).astype(jnp.bfloat16)
        bcat[...] = land_b[...].astype(jnp.bfloat16)

    xb = x_ref[...].astype(jnp.bfloat16)
    u = lax.dot_general(xb, acatt[...], (((1,), (1,)), ((), ())),
                        preferred_element_type=jnp.float32)
    u = u + bv[0:1, :]
    o_ref[...] = jnp.dot(u.astype(jnp.bfloat16), bcat[...],
                         preferred_element_type=jnp.float32)


def kernel(x, weights, A_hot, B_hot, latent_scale, latent_bias, top_k):
    batch, seq, d_in = x.shape
    d_out = B_hot.shape[-1]
    n_tok = batch * seq
    x_flat = x.reshape(n_tok, d_in)
    a_t = jnp.transpose(A_hot, (0, 2, 1))

    bt = 1024
    grid = (n_tok // bt,)
    out = pl.pallas_call(
        _body,
        grid=grid,
        in_specs=[
            pl.BlockSpec(memory_space=pltpu.SMEM),
            pl.BlockSpec((1, weights.shape[0]), lambda t: (0, 0)),
            pl.BlockSpec((RANK, 1), lambda t: (0, 0)),
            pl.BlockSpec((1, RANK), lambda t: (0, 0)),
            pl.BlockSpec((bt, d_in), lambda t: (t, 0)),
            pl.BlockSpec(memory_space=pltpu.MemorySpace.HBM),
            pl.BlockSpec(memory_space=pltpu.MemorySpace.HBM),
        ],
        out_specs=pl.BlockSpec((bt, d_out), lambda t: (t, 0)),
        out_shape=jax.ShapeDtypeStruct((n_tok, d_out), jnp.float32),
        scratch_shapes=[
            pltpu.VMEM((CAT, d_in), jnp.float32),
            pltpu.VMEM((CAT, d_out), jnp.float32),
            pltpu.VMEM((CAT, d_in), jnp.bfloat16),
            pltpu.VMEM((CAT, d_out), jnp.bfloat16),
            pltpu.VMEM((1, CAT), jnp.float32),
            pltpu.SemaphoreType.DMA((2 * TOPK,)),
        ],
    )(
        jnp.asarray(top_k, jnp.int32).reshape(1),
        weights.reshape(1, -1),
        latent_scale.reshape(-1, 1),
        latent_bias.reshape(1, -1),
        x_flat,
        a_t,
        B_hot,
    )
    return out.reshape(batch, seq, d_out)
